# baseline (speedup 1.0000x reference)
import functools

import jax
import jax.numpy as jnp
from jax import lax
from jax.experimental import pallas as pl
from jax.experimental.pallas import tpu as pltpu
from jax.experimental.pallas import tpu_sc as plsc

N = 50000
E = 800000
H = 64
HALF = 32
NSUB = 1000
NG = 64
ODIM = 8

EW = 128
NWINP = 6272
EPAD = NWINP * EW
PW = 80
NS = 50016
PN = NS * HALF // 128

_MESH = plsc.VectorSubcoreMesh(core_axis_name="c", subcore_axis_name="s")


def _zero_zb(zb, nrows):
    @pl.loop(0, nrows)
    def _(i):
        zb[i, pl.ds(0, 16)] = jnp.zeros((16,), jnp.float32)
        zb[i, pl.ds(16, 16)] = jnp.zeros((16,), jnp.float32)


def _sc_agg1(x_flat, ei3):
    @functools.partial(
        pl.kernel,
        out_type=jax.ShapeDtypeStruct((2 * N,), jnp.float32),
        mesh=_MESH,
        compiler_params=pltpu.CompilerParams(use_tc_tiling_on_sc=False),
        scratch_types=[
            pltpu.VMEM((196, EW), jnp.int32),
            pltpu.VMEM((196, EW), jnp.int32),
            pltpu.VMEM((2, 4 * EW), jnp.float32),
            pltpu.VMEM((400,), jnp.float32),
            pltpu.VMEM_SHARED((N + 8,), jnp.float32),
            pltpu.SemaphoreType.DMA((2,)),
            pltpu.SemaphoreType.DMA((2,)),
        ],
    )
    def kern(x_hbm, ei_hbm, out_hbm, sidx, didx, rows, zb, acc, gsem, ssem):
        c = lax.axis_index("c")
        t = lax.axis_index("s")

        @pl.loop(0, 400, step=16)
        def _(i):
            zb[pl.ds(i, 16)] = jnp.zeros((16,), jnp.float32)

        @pl.loop(0, 8)
        def _(k):
            ch = t + 16 * k

            @pl.when(ch < 125)
            def _():
                pltpu.sync_copy(zb, acc.at[pl.ds(ch * 400, 400)])

        plsc.subcore_barrier()

        base = (c * 16 + t) * 196
        pltpu.sync_copy(ei_hbm.at[0, pl.ds(base, 196)], sidx)
        pltpu.sync_copy(ei_hbm.at[1, pl.ds(base, 196)], didx)

        def issue_g(m, b):
            for i in range(4):
                pltpu.async_copy(x_hbm.at[sidx.at[m * 4 + i]],
                                 rows.at[b, pl.ds(i * EW, EW)], gsem.at[b])

        def issue_s(m, b):
            for i in range(4):
                pltpu.async_copy(rows.at[b, pl.ds(i * EW, EW)],
                                 acc.at[didx.at[m * 4 + i]],
                                 ssem.at[b], add=True)

        def gwait(b):
            pltpu.make_async_copy(x_hbm.at[pl.ds(0, 4 * EW)], rows.at[b],
                                  gsem.at[b]).wait()

        def swait(b):
            pltpu.make_async_copy(rows.at[b], acc.at[pl.ds(0, 4 * EW)],
                                  ssem.at[b]).wait()

        issue_g(0, 0)

        @pl.loop(0, 48, step=2)
        def _(m):
            @pl.when(m > 0)
            def _():
                swait(1)
            issue_g(m + 1, 1)
            gwait(0)
            issue_s(m, 0)
            swait(0)
            issue_g(m + 2, 0)
            gwait(1)
            issue_s(m + 1, 1)

        swait(1)
        gwait(0)
        issue_s(48, 0)
        swait(0)

        plsc.subcore_barrier()

        @pl.loop(0, 8)
        def _(k):
            ch = t + 16 * k

            @pl.when(ch < 125)
            def _():
                pltpu.sync_copy(acc.at[pl.ds(ch * 400, 400)], zb)
                pltpu.sync_copy(zb, out_hbm.at[pl.ds(c * N + ch * 400, 400)])

    return kern(x_flat, ei3)


def _sc_agg_vec(h, ei3):
    @functools.partial(
        pl.kernel,
        out_type=jax.ShapeDtypeStruct((2, NS, HALF), jnp.float32),
        mesh=_MESH,
        compiler_params=pltpu.CompilerParams(use_tc_tiling_on_sc=False),
        scratch_types=[
            pltpu.VMEM((56, EW), jnp.int32),
            pltpu.VMEM((56, EW), jnp.int32),
            pltpu.VMEM((2, 2 * EW, HALF), jnp.float32),
            pltpu.VMEM_SHARED((NS, HALF), jnp.float32),
            pltpu.SemaphoreType.DMA((2,)),
            pltpu.SemaphoreType.DMA((2,)),
        ],
    )
    def kern(h_hbm, ei_hbm, out_hbm, sidx, didx, rows, acc, gsem, ssem):
        c = lax.axis_index("c")
        t = lax.axis_index("s")
        @pl.loop(0, 2 * EW)
        def _(i):
            rows[0, i, pl.ds(0, 16)] = jnp.zeros((16,), jnp.float32)
            rows[0, i, pl.ds(16, 16)] = jnp.zeros((16,), jnp.float32)

        zb = rows.at[0, pl.ds(0, 200)]

        @pl.loop(0, 16)
        def _(k):
            ch = t + 16 * k

            @pl.when(ch < 250)
            def _():
                pltpu.sync_copy(zb, acc.at[pl.ds(ch * 200, 200)])

        plsc.subcore_barrier()
        hc = h_hbm.at[c]

        def issue_g(m, b):
            for i in range(2):
                pltpu.async_copy(hc.at[sidx.at[m * 2 + i]],
                                 rows.at[b, pl.ds(i * EW, EW)], gsem.at[b])

        def issue_s(m, b):
            for i in range(2):
                pltpu.async_copy(rows.at[b, pl.ds(i * EW, EW)],
                                 acc.at[didx.at[m * 2 + i]],
                                 ssem.at[b], add=True)

        def gwait(b):
            pltpu.make_async_copy(hc.at[pl.ds(0, 2 * EW)], rows.at[b],
                                  gsem.at[b]).wait()

        def swait(b):
            pltpu.make_async_copy(rows.at[b], acc.at[pl.ds(0, 2 * EW)],
                                  ssem.at[b]).wait()

        base = t * 392

        @pl.loop(0, 7)
        def _(sb):
            sbase = base + sb * 56
            pltpu.sync_copy(ei_hbm.at[0, pl.ds(sbase, 56)], sidx)
            pltpu.sync_copy(ei_hbm.at[1, pl.ds(sbase, 56)], didx)
            issue_g(0, 0)

            @pl.loop(0, 26, step=2)
            def _(m):
                @pl.when(m > 0)
                def _():
                    swait(1)
                issue_g(m + 1, 1)
                gwait(0)
                issue_s(m, 0)
                swait(0)
                issue_g(m + 2, 0)
                gwait(1)
                issue_s(m + 1, 1)

            swait(1)
            issue_g(27, 1)
            gwait(0)
            issue_s(26, 0)
            gwait(1)
            issue_s(27, 1)
            swait(0)
            swait(1)

        plsc.subcore_barrier()

        @pl.loop(0, 16)
        def _(k):
            ch = t + 16 * k

            @pl.when(ch < 250)
            def _():
                pltpu.sync_copy(acc.at[pl.ds(ch * 200, 200)], zb)
                pltpu.sync_copy(zb, out_hbm.at[c, pl.ds(ch * 200, 200)])

    return kern(h, ei3)


def _sc_pool(h, n2s2, s2g):
    @functools.partial(
        pl.kernel,
        out_type=jax.ShapeDtypeStruct((2, NG, HALF), jnp.float32),
        mesh=_MESH,
        compiler_params=pltpu.CompilerParams(use_tc_tiling_on_sc=False),
        scratch_types=[
            pltpu.VMEM((39, PW), jnp.int32),
            pltpu.VMEM((1, 40), jnp.int32),
            pltpu.VMEM((2, PW, HALF), jnp.float32),
            pltpu.VMEM((PW, HALF), jnp.float32),
            pltpu.VMEM_SHARED((NSUB, HALF), jnp.float32),
            pltpu.VMEM_SHARED((NG, HALF), jnp.float32),
            pltpu.SemaphoreType.DMA((2,)),
            pltpu.SemaphoreType.DMA((2,)),
        ],
    )
    def kern(h_hbm, n2s_hbm, s2g_hbm, out_hbm, idxb, idx2b, rows, zb,
             acc1, acc2, gsem, ssem):
        c = lax.axis_index("c")
        t = lax.axis_index("s")
        _zero_zb(zb, PW)

        @pl.loop(0, 2)
        def _(k):
            ch = t + 16 * k

            @pl.when(ch < 25)
            def _():
                pltpu.sync_copy(zb.at[pl.ds(0, 40)],
                                acc1.at[pl.ds(ch * 40, 40)])

        @pl.when(t == 15)
        def _():
            pltpu.sync_copy(zb.at[pl.ds(0, 64)], acc2)

        plsc.subcore_barrier()
        hc = h_hbm.at[c]

        base = t * 39
        pltpu.sync_copy(n2s_hbm.at[pl.ds(base, 39)], idxb)

        def issue_g(k, b):
            pltpu.async_copy(hc.at[pl.ds((base + k) * PW, PW)],
                             rows.at[b], gsem.at[b])

        def issue_s(k, b):
            pltpu.async_copy(rows.at[b], acc1.at[idxb.at[k]],
                             ssem.at[b], add=True)

        def gwait(b):
            pltpu.make_async_copy(hc.at[pl.ds(0, PW)], rows.at[b],
                                  gsem.at[b]).wait()

        def swait(b):
            pltpu.make_async_copy(rows.at[b], acc1.at[pl.ds(0, PW)],
                                  ssem.at[b]).wait()

        issue_g(0, 0)

        @pl.loop(0, 38, step=2)
        def _(k):
            @pl.when(k > 0)
            def _():
                swait(1)
            issue_g(k + 1, 1)
            gwait(0)
            issue_s(k, 0)
            swait(0)
            issue_g(k + 2, 0)
            gwait(1)
            issue_s(k + 1, 1)

        swait(1)
        gwait(0)
        issue_s(38, 0)
        swait(0)

        @pl.when(t == 0)
        def _():
            pltpu.sync_copy(n2s_hbm.at[pl.ds(624, 1)],
                            idxb.at[pl.ds(0, 1)])
            pltpu.sync_copy(hc.at[pl.ds(624 * PW, PW)], rows.at[0])
            pltpu.sync_copy(rows.at[0], acc1.at[idxb.at[0]], add=True)

        plsc.subcore_barrier()

        @pl.loop(0, 2)
        def _(k):
            w = t + 16 * k

            @pl.when(w < 25)
            def _():
                off = w * 40
                pltpu.sync_copy(s2g_hbm.at[pl.ds(off, 40)], idx2b.at[0])
                pltpu.sync_copy(acc1.at[pl.ds(off, 40)],
                                rows.at[0, pl.ds(0, 40)])
                pltpu.sync_copy(rows.at[0, pl.ds(0, 40)],
                                acc2.at[idx2b.at[0]], add=True)

        plsc.subcore_barrier()

        @pl.when(t == 0)
        def _():
            pltpu.sync_copy(acc2, zb.at[pl.ds(0, 64)])
            pltpu.sync_copy(zb.at[pl.ds(0, 64)], out_hbm.at[c])

    return kern(h, n2s2, s2g)


BX = 4168


def _tc_mlp1(x, p0, p1, W1, b1, W2, b2):
    def body(x_ref, p0_ref, p1_ref, w1_ref, b1_ref, w2_ref, b2_ref, o_ref):
        z = x_ref[...] + p0_ref[...] + p1_ref[...]
        a = jnp.maximum(z * w1_ref[...] + b1_ref[...], 0.0)
        v = jnp.dot(a, w2_ref[...],
                    preferred_element_type=jnp.float32) + b2_ref[...]
        v = jnp.maximum(v, 0.0)
        o_ref[0] = v[:, :HALF]
        o_ref[1] = v[:, HALF:]

    return pl.pallas_call(
        body,
        grid=(NS // BX,),
        in_specs=[
            pl.BlockSpec((BX, 1), lambda i: (i, 0)),
            pl.BlockSpec((BX, 1), lambda i: (i, 0)),
            pl.BlockSpec((BX, 1), lambda i: (i, 0)),
            pl.BlockSpec((1, H), lambda i: (0, 0)),
            pl.BlockSpec((1, H), lambda i: (0, 0)),
            pl.BlockSpec((H, H), lambda i: (0, 0)),
            pl.BlockSpec((1, H), lambda i: (0, 0)),
        ],
        out_specs=pl.BlockSpec((2, BX, HALF), lambda i: (0, i, 0)),
        out_shape=jax.ShapeDtypeStruct((2, NS, HALF), jnp.float32),
    )(x, p0, p1, W1, b1.reshape(1, H), W2, b2.reshape(1, H))


BP = 4168


def _tc_mlp(h_p, agg_p, W1, b1, W2, b2):
    e4 = jnp.eye(4, dtype=jnp.float32)
    W1L = jnp.kron(e4, W1[:HALF, :])
    W1H = jnp.kron(e4, W1[HALF:, :])
    W2P = jnp.kron(e4, W2)
    b1P = jnp.tile(b1, 4).reshape(1, 4 * H)
    b2P = jnp.tile(b2, 4).reshape(1, 4 * H)

    def body(h_ref, a_ref, w1l, w1h, w2p, b1r, b2r, o_ref):
        zl = h_ref[0] + a_ref[0]
        zh = h_ref[1] + a_ref[1]
        u = jnp.maximum(
            jnp.dot(zl, w1l[...], preferred_element_type=jnp.float32)
            + jnp.dot(zh, w1h[...], preferred_element_type=jnp.float32)
            + b1r[...], 0.0)
        v = jnp.maximum(
            jnp.dot(u, w2p[...], preferred_element_type=jnp.float32)
            + b2r[...], 0.0)
        o_ref[0] = jnp.concatenate(
            [v[:, 0:32], v[:, 64:96], v[:, 128:160], v[:, 192:224]], axis=1)
        o_ref[1] = jnp.concatenate(
            [v[:, 32:64], v[:, 96:128], v[:, 160:192], v[:, 224:256]], axis=1)

    spec3 = pl.BlockSpec((2, BP, 128), lambda i: (0, i, 0))
    wspec = lambda r, c: pl.BlockSpec((r, c), lambda i: (0, 0))
    return pl.pallas_call(
        body,
        grid=(PN // BP,),
        in_specs=[
            spec3,
            spec3,
            wspec(128, 256),
            wspec(128, 256),
            wspec(256, 256),
            wspec(1, 256),
            wspec(1, 256),
        ],
        out_specs=spec3,
        out_shape=jax.ShapeDtypeStruct((2, PN, 128), jnp.float32),
    )(h_p, agg_p, W1L, W1H, W2P, b1P, b2P)


def _tc_head(g, lin1_w, lin1_b, lin2_w, lin2_b):
    def body(g_ref, w1_ref, b1_ref, w2_ref, b2_ref, o_ref):
        gg = jnp.concatenate([g_ref[0], g_ref[1]], axis=1)
        a = jnp.maximum(
            jnp.dot(gg, w1_ref[...], preferred_element_type=jnp.float32)
            + b1_ref[...], 0.0)
        o = jnp.dot(a, w2_ref[...],
                    preferred_element_type=jnp.float32) + b2_ref[...]
        m = jnp.max(o, axis=1, keepdims=True)
        s = jnp.log(jnp.sum(jnp.exp(o - m), axis=1, keepdims=True))
        o_ref[...] = o - m - s

    return pl.pallas_call(
        body,
        out_shape=jax.ShapeDtypeStruct((NG, ODIM), jnp.float32),
    )(g, lin1_w, lin1_b.reshape(1, H), lin2_w, lin2_b.reshape(1, ODIM))


def kernel(x, params, edge_index, node_to_subgraph, subgraph_to_graph):
    mlps = params["mlps"]
    npad = EPAD - E
    ei = edge_index.astype(jnp.int32)
    ei_flat = jnp.concatenate([
        ei[0], jnp.zeros((npad,), jnp.int32),
        ei[1], N + (jnp.arange(npad, dtype=jnp.int32) % 8)])
    ei3 = ei_flat.reshape(2, NWINP, EW)
    agg1 = _sc_agg1(x.reshape(N), ei3)
    W1, b1, W2, b2 = mlps[0]
    h_sc = _tc_mlp1(x, agg1[:N].reshape(N, 1), agg1[N:].reshape(N, 1),
                    W1, b1, W2, b2)
    h_p = h_sc.reshape(2, PN, 128)
    for li in range(1, 4):
        W1, b1, W2, b2 = mlps[li]
        agg = _sc_agg_vec(h_p.reshape(2, NS, HALF), ei3)
        h_p = _tc_mlp(h_p, agg.reshape(2, PN, 128), W1, b1, W2, b2)
    g = _sc_pool(h_p.reshape(2, NS, HALF),
                 node_to_subgraph.reshape(625, PW), subgraph_to_graph)
    return _tc_head(g, params["lin1_w"], params["lin1_b"],
                    params["lin2_w"], params["lin2_b"])


# speedup vs baseline: 1.0133x; 1.0133x over previous
import functools

import jax
import jax.numpy as jnp
from jax import lax
from jax.experimental import pallas as pl
from jax.experimental.pallas import tpu as pltpu
from jax.experimental.pallas import tpu_sc as plsc

N = 50000
E = 800000
H = 64
HALF = 32
NSUB = 1000
NG = 64
ODIM = 8

EW = 128
NWINP = 6272
EPAD = NWINP * EW
PW = 80
NS = 50016
PN = NS * HALF // 128

_MESH = plsc.VectorSubcoreMesh(core_axis_name="c", subcore_axis_name="s")


def _zero_zb(zb, nrows):
    @pl.loop(0, nrows)
    def _(i):
        zb[i, pl.ds(0, 16)] = jnp.zeros((16,), jnp.float32)
        zb[i, pl.ds(16, 16)] = jnp.zeros((16,), jnp.float32)


def _sc_agg1(x_flat, ei3):
    @functools.partial(
        pl.kernel,
        out_type=jax.ShapeDtypeStruct((2 * N,), jnp.float32),
        mesh=_MESH,
        compiler_params=pltpu.CompilerParams(use_tc_tiling_on_sc=False),
        scratch_types=[
            pltpu.VMEM((196, EW), jnp.int32),
            pltpu.VMEM((196, EW), jnp.int32),
            pltpu.VMEM((2, 4 * EW), jnp.float32),
            pltpu.VMEM((400,), jnp.float32),
            pltpu.VMEM_SHARED((N + 8,), jnp.float32),
            pltpu.SemaphoreType.DMA((2,)),
            pltpu.SemaphoreType.DMA((2,)),
        ],
    )
    def kern(x_hbm, ei_hbm, out_hbm, sidx, didx, rows, zb, acc, gsem, ssem):
        c = lax.axis_index("c")
        t = lax.axis_index("s")

        @pl.loop(0, 400, step=16)
        def _(i):
            zb[pl.ds(i, 16)] = jnp.zeros((16,), jnp.float32)

        @pl.loop(0, 8)
        def _(k):
            ch = t + 16 * k

            @pl.when(ch < 125)
            def _():
                pltpu.sync_copy(zb, acc.at[pl.ds(ch * 400, 400)])

        plsc.subcore_barrier()

        base = (c * 16 + t) * 196
        pltpu.sync_copy(ei_hbm.at[0, pl.ds(base, 196)], sidx)
        pltpu.sync_copy(ei_hbm.at[1, pl.ds(base, 196)], didx)

        def issue_g(m, b):
            for i in range(4):
                pltpu.async_copy(x_hbm.at[sidx.at[m * 4 + i]],
                                 rows.at[b, pl.ds(i * EW, EW)], gsem.at[b])

        def issue_s(m, b):
            for i in range(4):
                pltpu.async_copy(rows.at[b, pl.ds(i * EW, EW)],
                                 acc.at[didx.at[m * 4 + i]],
                                 ssem.at[b], add=True)

        def gwait(b):
            pltpu.make_async_copy(x_hbm.at[pl.ds(0, 4 * EW)], rows.at[b],
                                  gsem.at[b]).wait()

        def swait(b):
            pltpu.make_async_copy(rows.at[b], acc.at[pl.ds(0, 4 * EW)],
                                  ssem.at[b]).wait()

        issue_g(0, 0)

        @pl.loop(0, 48, step=2)
        def _(m):
            @pl.when(m > 0)
            def _():
                swait(1)
            issue_g(m + 1, 1)
            gwait(0)
            issue_s(m, 0)
            swait(0)
            issue_g(m + 2, 0)
            gwait(1)
            issue_s(m + 1, 1)

        swait(1)
        gwait(0)
        issue_s(48, 0)
        swait(0)

        plsc.subcore_barrier()

        @pl.loop(0, 8)
        def _(k):
            ch = t + 16 * k

            @pl.when(ch < 125)
            def _():
                pltpu.sync_copy(acc.at[pl.ds(ch * 400, 400)], zb)
                pltpu.sync_copy(zb, out_hbm.at[pl.ds(c * N + ch * 400, 400)])

    return kern(x_flat, ei3)


def _sc_agg_vec(h, ei3):
    @functools.partial(
        pl.kernel,
        out_type=jax.ShapeDtypeStruct((2, NS, HALF), jnp.float32),
        mesh=_MESH,
        compiler_params=pltpu.CompilerParams(use_tc_tiling_on_sc=False),
        scratch_types=[
            pltpu.VMEM((56, EW), jnp.int32),
            pltpu.VMEM((56, EW), jnp.int32),
            pltpu.VMEM((2, 2 * EW, HALF), jnp.float32),
            pltpu.VMEM_SHARED((NS, HALF), jnp.float32),
            pltpu.SemaphoreType.DMA((2,)),
            pltpu.SemaphoreType.DMA((2,)),
            pltpu.SemaphoreType.DMA,
            pltpu.SemaphoreType.DMA((2,)),
        ],
    )
    def kern(h_hbm, ei_hbm, out_hbm, sidx, didx, rows, acc, gsem, ssem,
             zsem, dsem):
        c = lax.axis_index("c")
        t = lax.axis_index("s")
        @pl.loop(0, 2 * EW)
        def _(i):
            rows[0, i, pl.ds(0, 16)] = jnp.zeros((16,), jnp.float32)
            rows[0, i, pl.ds(16, 16)] = jnp.zeros((16,), jnp.float32)

        zb = rows.at[0, pl.ds(0, 200)]

        @pl.loop(0, 16)
        def _(k):
            ch = t + 16 * k

            @pl.when(ch < 250)
            def _():
                pltpu.async_copy(zb, acc.at[pl.ds(ch * 200, 200)], zsem)

        @pl.loop(0, 16)
        def _(k):
            ch = t + 16 * k

            @pl.when(ch < 250)
            def _():
                pltpu.make_async_copy(zb, acc.at[pl.ds(0, 200)],
                                      zsem).wait()

        plsc.subcore_barrier()
        hc = h_hbm.at[c]

        def issue_g(m, b):
            for i in range(2):
                pltpu.async_copy(hc.at[sidx.at[m * 2 + i]],
                                 rows.at[b, pl.ds(i * EW, EW)], gsem.at[b])

        def issue_s(m, b):
            for i in range(2):
                pltpu.async_copy(rows.at[b, pl.ds(i * EW, EW)],
                                 acc.at[didx.at[m * 2 + i]],
                                 ssem.at[b], add=True)

        def gwait(b):
            pltpu.make_async_copy(hc.at[pl.ds(0, 2 * EW)], rows.at[b],
                                  gsem.at[b]).wait()

        def swait(b):
            pltpu.make_async_copy(rows.at[b], acc.at[pl.ds(0, 2 * EW)],
                                  ssem.at[b]).wait()

        base = t * 392

        @pl.loop(0, 7)
        def _(sb):
            sbase = base + sb * 56
            pltpu.sync_copy(ei_hbm.at[0, pl.ds(sbase, 56)], sidx)
            pltpu.sync_copy(ei_hbm.at[1, pl.ds(sbase, 56)], didx)
            issue_g(0, 0)

            @pl.loop(0, 26, step=2)
            def _(m):
                @pl.when(m > 0)
                def _():
                    swait(1)
                issue_g(m + 1, 1)
                gwait(0)
                issue_s(m, 0)
                swait(0)
                issue_g(m + 2, 0)
                gwait(1)
                issue_s(m + 1, 1)

            swait(1)
            issue_g(27, 1)
            gwait(0)
            issue_s(26, 0)
            gwait(1)
            issue_s(27, 1)
            swait(0)
            swait(1)

        plsc.subcore_barrier()

        @pl.loop(0, 8)
        def _(k):
            for b in range(2):
                ch = t + 16 * (2 * k + b)

                @pl.when(ch < 250)
                def _():
                    @pl.when(k > 0)
                    def _():
                        pltpu.make_async_copy(
                            rows.at[b, pl.ds(0, 200)],
                            out_hbm.at[0, pl.ds(0, 200)], dsem.at[b]).wait()
                    pltpu.sync_copy(acc.at[pl.ds(ch * 200, 200)],
                                    rows.at[b, pl.ds(0, 200)])
                    pltpu.async_copy(rows.at[b, pl.ds(0, 200)],
                                     out_hbm.at[c, pl.ds(ch * 200, 200)],
                                     dsem.at[b])

        for b in range(2):
            pltpu.make_async_copy(rows.at[b, pl.ds(0, 200)],
                                  out_hbm.at[0, pl.ds(0, 200)],
                                  dsem.at[b]).wait()

    return kern(h, ei3)


def _sc_pool(h, n2s2, s2g):
    @functools.partial(
        pl.kernel,
        out_type=jax.ShapeDtypeStruct((2, NG, HALF), jnp.float32),
        mesh=_MESH,
        compiler_params=pltpu.CompilerParams(use_tc_tiling_on_sc=False),
        scratch_types=[
            pltpu.VMEM((39, PW), jnp.int32),
            pltpu.VMEM((1, 40), jnp.int32),
            pltpu.VMEM((2, PW, HALF), jnp.float32),
            pltpu.VMEM((PW, HALF), jnp.float32),
            pltpu.VMEM_SHARED((NSUB, HALF), jnp.float32),
            pltpu.VMEM_SHARED((NG, HALF), jnp.float32),
            pltpu.SemaphoreType.DMA((2,)),
            pltpu.SemaphoreType.DMA((2,)),
        ],
    )
    def kern(h_hbm, n2s_hbm, s2g_hbm, out_hbm, idxb, idx2b, rows, zb,
             acc1, acc2, gsem, ssem):
        c = lax.axis_index("c")
        t = lax.axis_index("s")
        _zero_zb(zb, PW)

        @pl.loop(0, 2)
        def _(k):
            ch = t + 16 * k

            @pl.when(ch < 25)
            def _():
                pltpu.sync_copy(zb.at[pl.ds(0, 40)],
                                acc1.at[pl.ds(ch * 40, 40)])

        @pl.when(t == 15)
        def _():
            pltpu.sync_copy(zb.at[pl.ds(0, 64)], acc2)

        plsc.subcore_barrier()
        hc = h_hbm.at[c]

        base = t * 39
        pltpu.sync_copy(n2s_hbm.at[pl.ds(base, 39)], idxb)

        def issue_g(k, b):
            pltpu.async_copy(hc.at[pl.ds((base + k) * PW, PW)],
                             rows.at[b], gsem.at[b])

        def issue_s(k, b):
            pltpu.async_copy(rows.at[b], acc1.at[idxb.at[k]],
                             ssem.at[b], add=True)

        def gwait(b):
            pltpu.make_async_copy(hc.at[pl.ds(0, PW)], rows.at[b],
                                  gsem.at[b]).wait()

        def swait(b):
            pltpu.make_async_copy(rows.at[b], acc1.at[pl.ds(0, PW)],
                                  ssem.at[b]).wait()

        issue_g(0, 0)

        @pl.loop(0, 38, step=2)
        def _(k):
            @pl.when(k > 0)
            def _():
                swait(1)
            issue_g(k + 1, 1)
            gwait(0)
            issue_s(k, 0)
            swait(0)
            issue_g(k + 2, 0)
            gwait(1)
            issue_s(k + 1, 1)

        swait(1)
        gwait(0)
        issue_s(38, 0)
        swait(0)

        @pl.when(t == 0)
        def _():
            pltpu.sync_copy(n2s_hbm.at[pl.ds(624, 1)],
                            idxb.at[pl.ds(0, 1)])
            pltpu.sync_copy(hc.at[pl.ds(624 * PW, PW)], rows.at[0])
            pltpu.sync_copy(rows.at[0], acc1.at[idxb.at[0]], add=True)

        plsc.subcore_barrier()

        @pl.loop(0, 2)
        def _(k):
            w = t + 16 * k

            @pl.when(w < 25)
            def _():
                off = w * 40
                pltpu.sync_copy(s2g_hbm.at[pl.ds(off, 40)], idx2b.at[0])
                pltpu.sync_copy(acc1.at[pl.ds(off, 40)],
                                rows.at[0, pl.ds(0, 40)])
                pltpu.sync_copy(rows.at[0, pl.ds(0, 40)],
                                acc2.at[idx2b.at[0]], add=True)

        plsc.subcore_barrier()

        @pl.when(t == 0)
        def _():
            pltpu.sync_copy(acc2, zb.at[pl.ds(0, 64)])
            pltpu.sync_copy(zb.at[pl.ds(0, 64)], out_hbm.at[c])

    return kern(h, n2s2, s2g)


BX = 4168


def _tc_mlp1(x, p0, p1, W1, b1, W2, b2):
    def body(x_ref, p0_ref, p1_ref, w1_ref, b1_ref, w2_ref, b2_ref, o_ref):
        z = x_ref[...] + p0_ref[...] + p1_ref[...]
        a = jnp.maximum(z * w1_ref[...] + b1_ref[...], 0.0)
        v = jnp.dot(a, w2_ref[...],
                    preferred_element_type=jnp.float32) + b2_ref[...]
        v = jnp.maximum(v, 0.0)
        o_ref[0] = v[:, :HALF]
        o_ref[1] = v[:, HALF:]

    return pl.pallas_call(
        body,
        grid=(NS // BX,),
        in_specs=[
            pl.BlockSpec((BX, 1), lambda i: (i, 0)),
            pl.BlockSpec((BX, 1), lambda i: (i, 0)),
            pl.BlockSpec((BX, 1), lambda i: (i, 0)),
            pl.BlockSpec((1, H), lambda i: (0, 0)),
            pl.BlockSpec((1, H), lambda i: (0, 0)),
            pl.BlockSpec((H, H), lambda i: (0, 0)),
            pl.BlockSpec((1, H), lambda i: (0, 0)),
        ],
        out_specs=pl.BlockSpec((2, BX, HALF), lambda i: (0, i, 0)),
        out_shape=jax.ShapeDtypeStruct((2, NS, HALF), jnp.float32),
    )(x, p0, p1, W1, b1.reshape(1, H), W2, b2.reshape(1, H))


BP = 4168


def _tc_mlp(h_p, agg_p, W1, b1, W2, b2):
    e4 = jnp.eye(4, dtype=jnp.float32)
    W1L = jnp.kron(e4, W1[:HALF, :])
    W1H = jnp.kron(e4, W1[HALF:, :])
    W2P = jnp.kron(e4, W2)
    b1P = jnp.tile(b1, 4).reshape(1, 4 * H)
    b2P = jnp.tile(b2, 4).reshape(1, 4 * H)

    def body(h_ref, a_ref, w1l, w1h, w2p, b1r, b2r, o_ref):
        zl = h_ref[0] + a_ref[0]
        zh = h_ref[1] + a_ref[1]
        u = jnp.maximum(
            jnp.dot(zl, w1l[...], preferred_element_type=jnp.float32)
            + jnp.dot(zh, w1h[...], preferred_element_type=jnp.float32)
            + b1r[...], 0.0)
        v = jnp.maximum(
            jnp.dot(u, w2p[...], preferred_element_type=jnp.float32)
            + b2r[...], 0.0)
        o_ref[0] = jnp.concatenate(
            [v[:, 0:32], v[:, 64:96], v[:, 128:160], v[:, 192:224]], axis=1)
        o_ref[1] = jnp.concatenate(
            [v[:, 32:64], v[:, 96:128], v[:, 160:192], v[:, 224:256]], axis=1)

    spec3 = pl.BlockSpec((2, BP, 128), lambda i: (0, i, 0))
    wspec = lambda r, c: pl.BlockSpec((r, c), lambda i: (0, 0))
    return pl.pallas_call(
        body,
        grid=(PN // BP,),
        in_specs=[
            spec3,
            spec3,
            wspec(128, 256),
            wspec(128, 256),
            wspec(256, 256),
            wspec(1, 256),
            wspec(1, 256),
        ],
        out_specs=spec3,
        out_shape=jax.ShapeDtypeStruct((2, PN, 128), jnp.float32),
    )(h_p, agg_p, W1L, W1H, W2P, b1P, b2P)


def _tc_head(g, lin1_w, lin1_b, lin2_w, lin2_b):
    def body(g_ref, w1_ref, b1_ref, w2_ref, b2_ref, o_ref):
        gg = jnp.concatenate([g_ref[0], g_ref[1]], axis=1)
        a = jnp.maximum(
            jnp.dot(gg, w1_ref[...], preferred_element_type=jnp.float32)
            + b1_ref[...], 0.0)
        o = jnp.dot(a, w2_ref[...],
                    preferred_element_type=jnp.float32) + b2_ref[...]
        m = jnp.max(o, axis=1, keepdims=True)
        s = jnp.log(jnp.sum(jnp.exp(o - m), axis=1, keepdims=True))
        o_ref[...] = o - m - s

    return pl.pallas_call(
        body,
        out_shape=jax.ShapeDtypeStruct((NG, ODIM), jnp.float32),
    )(g, lin1_w, lin1_b.reshape(1, H), lin2_w, lin2_b.reshape(1, ODIM))


def kernel(x, params, edge_index, node_to_subgraph, subgraph_to_graph):
    mlps = params["mlps"]
    npad = EPAD - E
    ei = edge_index.astype(jnp.int32)
    ei_flat = jnp.concatenate([
        ei[0], jnp.zeros((npad,), jnp.int32),
        ei[1], N + (jnp.arange(npad, dtype=jnp.int32) % 8)])
    ei3 = ei_flat.reshape(2, NWINP, EW)
    agg1 = _sc_agg1(x.reshape(N), ei3)
    W1, b1, W2, b2 = mlps[0]
    h_sc = _tc_mlp1(x, agg1[:N].reshape(N, 1), agg1[N:].reshape(N, 1),
                    W1, b1, W2, b2)
    h_p = h_sc.reshape(2, PN, 128)
    for li in range(1, 4):
        W1, b1, W2, b2 = mlps[li]
        agg = _sc_agg_vec(h_p.reshape(2, NS, HALF), ei3)
        h_p = _tc_mlp(h_p, agg.reshape(2, PN, 128), W1, b1, W2, b2)
    g = _sc_pool(h_p.reshape(2, NS, HALF),
                 node_to_subgraph.reshape(625, PW), subgraph_to_graph)
    return _tc_head(g, params["lin1_w"], params["lin1_b"],
                    params["lin2_w"], params["lin2_b"])
